# baseline (speedup 1.0000x reference)
import functools

import jax
import jax.numpy as jnp
from jax import lax
from jax.experimental import pallas as pl
from jax.experimental.pallas import tpu as pltpu
from jax.experimental.pallas import tpu_sc as plsc

N = 10000
E = 320000
D_IN = 128
HID = 16
D_OUT = 128

L = 16
NC = 2
NS = 16
NW = NC * NS
CHUNK = 128
NCHUNKS = E // CHUNK
PT = NCHUNKS // NW
XT = NCHUNKS - PT * NW
N_PAD = 10112
RPT = N_PAD // NS
RW = N_PAD * L // 128
RWX = N * L // 128

_mesh = plsc.VectorSubcoreMesh(core_axis_name="c", subcore_axis_name="s")
_f32 = jnp.float32
_sc_params = pltpu.CompilerParams(use_tc_tiling_on_sc=False)
_sc_deg_params = pltpu.CompilerParams(use_tc_tiling_on_sc=False,
                                      needs_layout_passes=False)


def _zero_accum(sid, stage, accum):
    @pl.loop(0, RPT)
    def _(i):
        stage.at[i][...] = jnp.zeros((L,), _f32)

    pltpu.sync_copy(stage, accum.at[pl.ds(sid * RPT, RPT)])


def _writeback(cid, sid, out_hbm, accum):
    plsc.subcore_barrier()
    pltpu.sync_copy(
        accum.at[pl.ds(sid * RPT, RPT)],
        out_hbm.at[cid].at[pl.ds(sid * RPT, RPT)],
    )


NPT = N_PAD // NS


@functools.partial(
    pl.kernel,
    out_type=jax.ShapeDtypeStruct((NC, N_PAD), jnp.int32),
    mesh=_mesh,
    compiler_params=_sc_deg_params,
    scratch_types=[
        pltpu.VMEM((PT + 1, CHUNK), jnp.int32),
        pltpu.VMEM((N_PAD,), jnp.int32),
        pltpu.VMEM((NS, NPT), jnp.int32),
        pltpu.VMEM_SHARED((NS, N_PAD), jnp.int32),
        pltpu.SemaphoreType.DMA,
    ],
)
def _sc_deg(ei_hbm, out_hbm, cbuf, hist, mbuf, hists, lsem):
    cid = lax.axis_index("c")
    sid = lax.axis_index("s")
    wid = sid * NC + cid
    base = wid * PT + jnp.minimum(wid, XT)
    extra = wid < XT

    hl = pltpu.async_copy(ei_hbm.at[1].at[pl.ds(base, PT)],
                          cbuf.at[pl.ds(0, PT)], lsem)

    @pl.when(extra)
    def _():
        pltpu.sync_copy(ei_hbm.at[1].at[pl.ds(base + PT, 1)],
                        cbuf.at[pl.ds(PT, 1)])

    @pl.loop(0, N_PAD, step=L)
    def _(i):
        hist.at[pl.ds(i, L)][...] = jnp.zeros((L,), jnp.int32)

    hl.wait()
    ones = jnp.ones((L,), jnp.int32)

    @pl.loop(0, PT)
    def _(j):
        @pl.loop(0, CHUNK, step=L)
        def _(k):
            iv = cbuf.at[j].at[pl.ds(k, L)][...]
            plsc.addupdate_scatter(hist, [iv], ones)

    @pl.when(extra)
    def _():
        @pl.loop(0, CHUNK, step=L)
        def _(k):
            iv = cbuf.at[PT].at[pl.ds(k, L)][...]
            plsc.addupdate_scatter(hist, [iv], ones)

    pltpu.sync_copy(hist, hists.at[sid])
    plsc.subcore_barrier()

    pltpu.sync_copy(hists.at[:, pl.ds(sid * NPT, NPT)], mbuf)

    @pl.loop(0, NPT, step=L)
    def _(v):
        acc = mbuf.at[0].at[pl.ds(v, L)][...]
        for t in range(1, NS):
            acc = acc + mbuf.at[t].at[pl.ds(v, L)][...]
        hist.at[pl.ds(v, L)][...] = acc

    pltpu.sync_copy(hist.at[pl.ds(0, NPT)],
                    out_hbm.at[cid].at[pl.ds(sid * NPT, NPT)])


K = 13
NG = PT // K


@functools.partial(
    pl.kernel,
    out_type=jax.ShapeDtypeStruct((NC, N_PAD, L), _f32),
    mesh=_mesh,
    compiler_params=_sc_params,
    scratch_types=[
        pltpu.VMEM((PT + 1, CHUNK), jnp.int32),
        pltpu.VMEM((PT + 1, CHUNK), jnp.int32),
        pltpu.VMEM((2, K, CHUNK, L), _f32),
        pltpu.VMEM((RPT, L), _f32),
        pltpu.VMEM_SHARED((N_PAD, L), _f32),
        pltpu.VMEM_SHARED((N_PAD, L), _f32),
        pltpu.SemaphoreType.DMA,
        pltpu.SemaphoreType.DMA,
        pltpu.SemaphoreType.DMA,
        pltpu.SemaphoreType.DMA,
        pltpu.SemaphoreType.DMA,
        pltpu.SemaphoreType.DMA,
        pltpu.SemaphoreType.DMA,
    ],
)
def _sc_agg(src_hbm, ei_hbm, out_hbm, rbuf, cbuf, msgs, stage,
            stable, accum, lsem, gsem0, gsem1, hsem0, hsem1, ssem0, ssem1):
    cid = lax.axis_index("c")
    sid = lax.axis_index("s")
    wid = sid * NC + cid
    base = wid * PT + jnp.minimum(wid, XT)
    extra = wid < XT
    gsems = (gsem0, gsem1)
    hsems = (hsem0, hsem1)
    ssems = (ssem0, ssem1)

    hr = pltpu.async_copy(ei_hbm.at[0].at[pl.ds(base, PT)],
                          rbuf.at[pl.ds(0, PT)], lsem)
    hc = pltpu.async_copy(ei_hbm.at[1].at[pl.ds(base, PT)],
                          cbuf.at[pl.ds(0, PT)], lsem)
    ht = pltpu.async_copy(src_hbm.at[pl.ds(sid * RPT, RPT)],
                          stable.at[pl.ds(sid * RPT, RPT)], lsem)

    @pl.when(extra)
    def _():
        pltpu.sync_copy(ei_hbm.at[0].at[pl.ds(base + PT, 1)],
                        rbuf.at[pl.ds(PT, 1)])
        pltpu.sync_copy(ei_hbm.at[1].at[pl.ds(base + PT, 1)],
                        cbuf.at[pl.ds(PT, 1)])

    _zero_accum(sid, stage, accum)
    hr.wait()
    hc.wait()
    ht.wait()
    plsc.subcore_barrier()

    def fire_gathers(g, s):
        hs = [pltpu.async_copy(src_hbm.at[rbuf.at[g * K + b]],
                               msgs.at[s].at[b], hsems[s])
              for b in range(K) if b % 3 == 2]
        vs = [pltpu.async_copy(stable.at[rbuf.at[g * K + b]],
                               msgs.at[s].at[b], gsems[s])
              for b in range(K) if b % 3 != 2]
        return hs + vs

    def fire_scatters(g, s):
        return [pltpu.async_copy(msgs.at[s].at[b],
                                 accum.at[cbuf.at[g * K + b]], ssems[s],
                                 add=True)
                for b in range(K)]

    gh = [None, None]
    sh = [None, None]
    gh[0] = fire_gathers(0, 0)
    for g in range(NG):
        s = g % 2
        if g + 1 < NG:
            if sh[1 - s] is not None:
                for h in sh[1 - s]:
                    h.wait()
            gh[1 - s] = fire_gathers(g + 1, 1 - s)
        for h in gh[s]:
            h.wait()
        sh[s] = fire_scatters(g, s)
    for hs in sh:
        if hs is not None:
            for h in hs:
                h.wait()

    @pl.when(extra)
    def _():
        pltpu.sync_copy(stable.at[rbuf.at[PT]], msgs.at[0].at[0])
        pltpu.sync_copy(msgs.at[0].at[0], accum.at[cbuf.at[PT]], add=True)

    _writeback(cid, sid, out_hbm, accum)


def _dot(a, b):
    return jnp.dot(a, b, precision=lax.Precision.DEFAULT,
                   preferred_element_type=_f32)


def _mm1_body(x_ref, w_ref, o_ref):
    o_ref[...] = jnp.zeros((RW, 128), _f32)
    h = _dot(x_ref[...], w_ref[...])
    h3 = h.reshape(RWX, 8, HID)
    for i in range(8):
        o_ref[pl.ds(0, RWX), pl.ds(i * HID, HID)] = h3[:, i, :]


def _prep_body(parts_ref, h_ref, rep_ref, dinv_ref, s_ref):
    deg = (parts_ref[0] + parts_ref[1] + 1).astype(_f32)
    dsm = lax.rsqrt(deg)
    for m in range(L):
        dinv_ref[:, m, :] = jnp.dot(dsm, rep_ref[m],
                                    precision=lax.Precision.HIGHEST,
                                    preferred_element_type=_f32)
    s_ref[...] = dinv_ref[...].reshape(RW, 128) * h_ref[...]


def _mid_body(parts_ref, s_ref, dinv_ref, b_ref, o_ref):
    dinv = dinv_ref[...]
    t = dinv * (parts_ref[0] + parts_ref[1] + s_ref[...]) + b_ref[...]
    o_ref[...] = dinv * jnp.maximum(t, 0.0)


def _fin_body(parts_ref, s_ref, dinv_ref, w_ref, b_ref, o_ref):
    q = dinv_ref[...] * (parts_ref[0] + parts_ref[1] + s_ref[...])
    o = _dot(q[:RWX, :], w_ref[...]) + b_ref[...]
    for i in range(8):
        o_ref[:, i, :] = o[:, i * D_OUT:(i + 1) * D_OUT]


_mm1 = pl.pallas_call(
    _mm1_body, out_shape=jax.ShapeDtypeStruct((RW, 128), _f32))
_prep = pl.pallas_call(
    _prep_body, out_shape=(jax.ShapeDtypeStruct((RW // L, L, 128), _f32),
                           jax.ShapeDtypeStruct((RW, 128), _f32)))
_mid = pl.pallas_call(
    _mid_body, out_shape=jax.ShapeDtypeStruct((RW, 128), _f32))
_fin = pl.pallas_call(
    _fin_body, out_shape=jax.ShapeDtypeStruct((RWX, 8, D_OUT), _f32))


def kernel(x, edge_index, W1, b1, W2, b2):
    ei = edge_index.astype(jnp.int32).reshape(2, NCHUNKS, CHUNK)

    h1 = _mm1(x, W1)
    deg_parts = _sc_deg(ei)
    rep = (jnp.arange(128)[None, :, None]
           == (8 * jnp.arange(L)[:, None, None]
               + jnp.arange(128)[None, None, :] // L)).astype(_f32)
    pw = (NC, RW, 128)
    dinv3, s1 = _prep(deg_parts.reshape(NC, RW // L, 128), h1, rep)
    dinv = dinv3.reshape(RW, 128)
    agg1 = _sc_agg(s1.reshape(N_PAD, L), ei)
    b1w = jnp.tile(b1, 128 // HID).reshape(1, 128)
    s2 = _mid(agg1.reshape(pw), s1, dinv, b1w)
    agg2 = _sc_agg(s2.reshape(N_PAD, L), ei)
    w2big = jax.scipy.linalg.block_diag(*([W2] * 8))
    b2big = jnp.tile(b2, 8).reshape(1, 8 * D_OUT)
    out = _fin(agg2.reshape(pw), s2, dinv, w2big, b2big)
    return out.reshape(N, D_OUT)


# speedup vs baseline: 1.0017x; 1.0017x over previous
import functools

import jax
import jax.numpy as jnp
from jax import lax
from jax.experimental import pallas as pl
from jax.experimental.pallas import tpu as pltpu
from jax.experimental.pallas import tpu_sc as plsc

N = 10000
E = 320000
D_IN = 128
HID = 16
D_OUT = 128

L = 16
NC = 2
NS = 16
NW = NC * NS
CHUNK = 128
NCHUNKS = E // CHUNK
PT = NCHUNKS // NW
XT = NCHUNKS - PT * NW
N_PAD = 10112
RPT = N_PAD // NS
RW = N_PAD * L // 128
RWX = N * L // 128

_mesh = plsc.VectorSubcoreMesh(core_axis_name="c", subcore_axis_name="s")
_f32 = jnp.float32
_sc_params = pltpu.CompilerParams(use_tc_tiling_on_sc=False)
_sc_deg_params = pltpu.CompilerParams(use_tc_tiling_on_sc=False,
                                      needs_layout_passes=False)


def _zero_accum(sid, stage, accum):
    @pl.loop(0, RPT)
    def _(i):
        stage.at[i][...] = jnp.zeros((L,), _f32)

    pltpu.sync_copy(stage, accum.at[pl.ds(sid * RPT, RPT)])


def _writeback(cid, sid, out_hbm, accum):
    plsc.subcore_barrier()
    pltpu.sync_copy(
        accum.at[pl.ds(sid * RPT, RPT)],
        out_hbm.at[cid].at[pl.ds(sid * RPT, RPT)],
    )


NPT = N_PAD // NS


@functools.partial(
    pl.kernel,
    out_type=jax.ShapeDtypeStruct((NC, N_PAD), jnp.int32),
    mesh=_mesh,
    compiler_params=_sc_deg_params,
    scratch_types=[
        pltpu.VMEM((PT + 1, CHUNK), jnp.int32),
        pltpu.VMEM((N_PAD,), jnp.int32),
        pltpu.VMEM((NS, NPT), jnp.int32),
        pltpu.VMEM_SHARED((NS, N_PAD), jnp.int32),
        pltpu.SemaphoreType.DMA,
    ],
)
def _sc_deg(ei_hbm, out_hbm, cbuf, hist, mbuf, hists, lsem):
    cid = lax.axis_index("c")
    sid = lax.axis_index("s")
    wid = sid * NC + cid
    base = wid * PT + jnp.minimum(wid, XT)
    extra = wid < XT

    hl = pltpu.async_copy(ei_hbm.at[1].at[pl.ds(base, PT)],
                          cbuf.at[pl.ds(0, PT)], lsem)

    @pl.when(extra)
    def _():
        pltpu.sync_copy(ei_hbm.at[1].at[pl.ds(base + PT, 1)],
                        cbuf.at[pl.ds(PT, 1)])

    @pl.loop(0, N_PAD, step=L)
    def _(i):
        hist.at[pl.ds(i, L)][...] = jnp.zeros((L,), jnp.int32)

    hl.wait()
    ones = jnp.ones((L,), jnp.int32)

    @pl.loop(0, PT)
    def _(j):
        @pl.loop(0, CHUNK, step=L)
        def _(k):
            iv = cbuf.at[j].at[pl.ds(k, L)][...]
            plsc.addupdate_scatter(hist, [iv], ones)

    @pl.when(extra)
    def _():
        @pl.loop(0, CHUNK, step=L)
        def _(k):
            iv = cbuf.at[PT].at[pl.ds(k, L)][...]
            plsc.addupdate_scatter(hist, [iv], ones)

    pltpu.sync_copy(hist, hists.at[sid])
    plsc.subcore_barrier()

    pltpu.sync_copy(hists.at[:, pl.ds(sid * NPT, NPT)], mbuf)

    @pl.loop(0, NPT, step=L)
    def _(v):
        acc = mbuf.at[0].at[pl.ds(v, L)][...]
        for t in range(1, NS):
            acc = acc + mbuf.at[t].at[pl.ds(v, L)][...]
        hist.at[pl.ds(v, L)][...] = acc

    pltpu.sync_copy(hist.at[pl.ds(0, NPT)],
                    out_hbm.at[cid].at[pl.ds(sid * NPT, NPT)])


K = 13
NG = PT // K


@functools.partial(
    pl.kernel,
    out_type=jax.ShapeDtypeStruct((NC, N_PAD, L), _f32),
    mesh=_mesh,
    compiler_params=_sc_params,
    scratch_types=[
        pltpu.VMEM((PT + 1, CHUNK), jnp.int32),
        pltpu.VMEM((PT + 1, CHUNK), jnp.int32),
        pltpu.VMEM((2, K, CHUNK, L), _f32),
        pltpu.VMEM((RPT, L), _f32),
        pltpu.VMEM_SHARED((N_PAD, L), _f32),
        pltpu.VMEM_SHARED((N_PAD, L), _f32),
        pltpu.SemaphoreType.DMA,
        pltpu.SemaphoreType.DMA,
        pltpu.SemaphoreType.DMA,
        pltpu.SemaphoreType.DMA,
        pltpu.SemaphoreType.DMA,
    ],
)
def _sc_agg(src_hbm, ei_hbm, out_hbm, rbuf, cbuf, msgs, stage,
            stable, accum, lsem, gsem0, gsem1, ssem0, ssem1):
    cid = lax.axis_index("c")
    sid = lax.axis_index("s")
    wid = sid * NC + cid
    base = wid * PT + jnp.minimum(wid, XT)
    extra = wid < XT
    gsems = (gsem0, gsem1)
    ssems = (ssem0, ssem1)

    hr = pltpu.async_copy(ei_hbm.at[0].at[pl.ds(base, PT)],
                          rbuf.at[pl.ds(0, PT)], lsem)
    hc = pltpu.async_copy(ei_hbm.at[1].at[pl.ds(base, PT)],
                          cbuf.at[pl.ds(0, PT)], lsem)
    ht = pltpu.async_copy(src_hbm.at[pl.ds(sid * RPT, RPT)],
                          stable.at[pl.ds(sid * RPT, RPT)], lsem)

    @pl.when(extra)
    def _():
        pltpu.sync_copy(ei_hbm.at[0].at[pl.ds(base + PT, 1)],
                        rbuf.at[pl.ds(PT, 1)])
        pltpu.sync_copy(ei_hbm.at[1].at[pl.ds(base + PT, 1)],
                        cbuf.at[pl.ds(PT, 1)])

    _zero_accum(sid, stage, accum)
    hr.wait()
    hc.wait()
    ht.wait()
    plsc.subcore_barrier()

    def fire_gathers(g, s):
        return [pltpu.async_copy(stable.at[rbuf.at[g * K + b]],
                                 msgs.at[s].at[b], gsems[s])
                for b in range(K)]

    def fire_scatters(g, s):
        return [pltpu.async_copy(msgs.at[s].at[b],
                                 accum.at[cbuf.at[g * K + b]], ssems[s],
                                 add=True)
                for b in range(K)]

    gh = [None, None]
    sh = [None, None]
    gh[0] = fire_gathers(0, 0)
    for g in range(NG):
        s = g % 2
        if g + 1 < NG:
            if sh[1 - s] is not None:
                for h in sh[1 - s]:
                    h.wait()
            gh[1 - s] = fire_gathers(g + 1, 1 - s)
        for h in gh[s]:
            h.wait()
        sh[s] = fire_scatters(g, s)
    for hs in sh:
        if hs is not None:
            for h in hs:
                h.wait()

    @pl.when(extra)
    def _():
        pltpu.sync_copy(stable.at[rbuf.at[PT]], msgs.at[0].at[0])
        pltpu.sync_copy(msgs.at[0].at[0], accum.at[cbuf.at[PT]], add=True)

    _writeback(cid, sid, out_hbm, accum)


def _dot(a, b):
    return jnp.dot(a, b, precision=lax.Precision.DEFAULT,
                   preferred_element_type=_f32)


def _mm1_body(x_ref, w_ref, o_ref):
    o_ref[...] = jnp.zeros((RW, 128), _f32)
    h = _dot(x_ref[...], w_ref[...])
    h3 = h.reshape(RWX, 8, HID)
    for i in range(8):
        o_ref[pl.ds(0, RWX), pl.ds(i * HID, HID)] = h3[:, i, :]


def _prep_body(parts_ref, h_ref, rep_ref, dinv_ref, s_ref):
    deg = (parts_ref[0] + parts_ref[1] + 1).astype(_f32)
    dsm = lax.rsqrt(deg)
    for m in range(L):
        dinv_ref[:, m, :] = jnp.dot(dsm, rep_ref[m],
                                    precision=lax.Precision.HIGHEST,
                                    preferred_element_type=_f32)
    s_ref[...] = dinv_ref[...].reshape(RW, 128) * h_ref[...]


def _mid_body(parts_ref, s_ref, dinv_ref, b_ref, o_ref):
    dinv = dinv_ref[...]
    t = dinv * (parts_ref[0] + parts_ref[1] + s_ref[...]) + b_ref[...]
    o_ref[...] = dinv * jnp.maximum(t, 0.0)


def _fin_body(parts_ref, s_ref, dinv_ref, w_ref, b_ref, o_ref):
    q = dinv_ref[...] * (parts_ref[0] + parts_ref[1] + s_ref[...])
    o = _dot(q[:RWX, :], w_ref[...]) + b_ref[...]
    for i in range(8):
        o_ref[:, i, :] = o[:, i * D_OUT:(i + 1) * D_OUT]


_mm1 = pl.pallas_call(
    _mm1_body, out_shape=jax.ShapeDtypeStruct((RW, 128), _f32))
_prep = pl.pallas_call(
    _prep_body, out_shape=(jax.ShapeDtypeStruct((RW // L, L, 128), _f32),
                           jax.ShapeDtypeStruct((RW, 128), _f32)))
_mid = pl.pallas_call(
    _mid_body, out_shape=jax.ShapeDtypeStruct((RW, 128), _f32))
_fin = pl.pallas_call(
    _fin_body, out_shape=jax.ShapeDtypeStruct((RWX, 8, D_OUT), _f32))


def kernel(x, edge_index, W1, b1, W2, b2):
    ei = edge_index.astype(jnp.int32).reshape(2, NCHUNKS, CHUNK)

    h1 = _mm1(x, W1)
    deg_parts = _sc_deg(ei)
    rep = (jnp.arange(128)[None, :, None]
           == (8 * jnp.arange(L)[:, None, None]
               + jnp.arange(128)[None, None, :] // L)).astype(_f32)
    pw = (NC, RW, 128)
    dinv3, s1 = _prep(deg_parts.reshape(NC, RW // L, 128), h1, rep)
    dinv = dinv3.reshape(RW, 128)
    agg1 = _sc_agg(s1.reshape(N_PAD, L), ei)
    b1w = jnp.tile(b1, 128 // HID).reshape(1, 128)
    s2 = _mid(agg1.reshape(pw), s1, dinv, b1w)
    agg2 = _sc_agg(s2.reshape(N_PAD, L), ei)
    w2big = jax.scipy.linalg.block_diag(*([W2] * 8))
    b2big = jnp.tile(b2, 8).reshape(1, 8 * D_OUT)
    out = _fin(agg2.reshape(pw), s2, dinv, w2big, b2big)
    return out.reshape(N, D_OUT)
